# baseline (speedup 1.0000x reference)
import functools

import jax
import jax.numpy as jnp
from jax import lax
from jax.experimental import pallas as pl
from jax.experimental.pallas import tpu as pltpu
from jax.experimental.pallas import tpu_sc as plsc

N_EMB = 1000000
DIM = 64
NCB = 8
CB_SIZE = 256
CB_DIM = 8
BATCH = 4096
HIST = 200
N = BATCH * HIST

NW = 32
NSUBC = 16
PER_W = N // NW
SUB = 64
CHUNK = 512
HALF = CHUNK // 2
NSUB = CHUNK // SUB
NCHUNK = PER_W // CHUNK
FSUB = 128
G2 = HALF * NCB // FSUB

STRIPE = 2000
NSTRIPE = N_EMB // STRIPE
TPS = (NSTRIPE + NSUBC - 1) // NSUBC


def _build_kernel():
    mesh = plsc.VectorSubcoreMesh(core_axis_name="c", subcore_axis_name="s")

    @functools.partial(
        pl.kernel,
        out_type=(jax.ShapeDtypeStruct((N * NCB, CB_DIM), jnp.float32),
                  jax.ShapeDtypeStruct((N_EMB, NCB), jnp.int32)),
        mesh=mesh,
        scratch_types=[
            pltpu.VMEM((2, NCB, STRIPE), jnp.int32),
            pltpu.VMEM((2, STRIPE, NCB), jnp.int32),
            pltpu.VMEM((2, NSUB, SUB), jnp.int32),
            pltpu.VMEM((2, CHUNK, NCB), jnp.int32),
            pltpu.VMEM((CHUNK * NCB,), jnp.int32),
            pltpu.VMEM((2, HALF * NCB, CB_DIM), jnp.float32),
            pltpu.VMEM_SHARED((NCB * CB_SIZE, CB_DIM), jnp.float32),
            pltpu.SemaphoreType.DMA,
            pltpu.SemaphoreType.DMA,
            pltpu.SemaphoreType.DMA,
            pltpu.SemaphoreType.DMA,
            pltpu.SemaphoreType.DMA,
            pltpu.SemaphoreType.DMA,
            pltpu.SemaphoreType.DMA,
            pltpu.SemaphoreType.DMA,
        ],
        compiler_params=pltpu.CompilerParams(
            needs_layout_passes=False, use_tc_tiling_on_sc=False),
    )
    def rpq_sc(idx_hbm, codes_hbm, cb_hbm, out_hbm, cp_hbm,
               pin_v, pout_v, idx_v, il_v, fl_v, out_v, cb_sh,
               sem_pi, sem_ps0, sem_ps1, sem_g, sem_c0, sem_c1,
               sem_o0, sem_o1):
        cid = lax.axis_index("c")
        sid = lax.axis_index("s")
        wid = cid * NSUBC + sid
        sem_ps = (sem_ps0, sem_ps1)
        sem_c = (sem_c0, sem_c1)
        sem_o = (sem_o0, sem_o1)
        iota16 = lax.iota(jnp.int32, 16)


        @pl.when(sid == 0)
        def _():
            pltpu.sync_copy(cb_hbm, cb_sh)

        def stripe_of(t):
            return jnp.minimum(t * NSUBC + sid, NSTRIPE - 1)

        def pfetch(t, slot):
            off = pl.multiple_of(stripe_of(t) * STRIPE, 8)
            pltpu.async_copy(codes_hbm.at[:, pl.ds(off, STRIPE)],
                             pin_v.at[slot], sem_pi)

        pfetch(0, 0)

        @pl.loop(0, TPS, step=2)
        def stripe_loop(t2):
            for dp in range(2):
                t = t2 + dp
                pltpu.make_async_copy(codes_hbm.at[:, pl.ds(0, STRIPE)],
                                      pin_v.at[dp], sem_pi).wait()

                @pl.when(t + 1 < TPS)
                def _():
                    pfetch(t + 1, 1 - dp)

                @pl.when(t >= 2)
                def _():
                    pltpu.make_async_copy(cp_hbm.at[pl.ds(0, STRIPE)],
                                          pout_v.at[dp], sem_ps[dp]).wait()

                pvec = jnp.broadcast_to(dp, (16,))

                @pl.loop(0, STRIPE // 16)
                def grp(i):
                    vvec = i * 16 + iota16
                    vals = []
                    for h in range(NCB):
                        hvec = jnp.full((16,), h, jnp.int32)
                        vals.append(
                            plsc.load_gather(pin_v, [pvec, hvec, vvec])
                            + (h * CB_SIZE))
                    for h in range(NCB):
                        hvec = jnp.full((16,), h, jnp.int32)
                        plsc.store_scatter(pout_v, [pvec, vvec, hvec],
                                           vals[h])

                off = pl.multiple_of(stripe_of(t) * STRIPE, 8)
                pltpu.async_copy(pout_v.at[dp], cp_hbm.at[pl.ds(off, STRIPE)],
                                 sem_ps[dp])

        for t in (TPS - 2, TPS - 1):
            pltpu.make_async_copy(cp_hbm.at[pl.ds(0, STRIPE)],
                                  pout_v.at[t % 2], sem_ps[t % 2]).wait()

        plsc.subcore_barrier()

        row_base = wid * PER_W
        sub_base = row_base // SUB

        def fetch(g, slot):
            sub_off = pl.multiple_of(sub_base + g * NSUB, 8)
            pltpu.sync_copy(idx_hbm.at[pl.ds(sub_off, NSUB)], idx_v.at[slot])
            for j in range(NSUB):
                pltpu.async_copy(cp_hbm.at[idx_v.at[slot, j]],
                                 il_v.at[slot, pl.ds(j * SUB, SUB)],
                                 sem_g)

        fetch(0, 0)

        @pl.loop(0, NCHUNK)
        def chunk_loop(g):
            p = lax.rem(g, 2)
            pltpu.make_async_copy(cp_hbm.at[pl.ds(0, CHUNK)],
                                  il_v.at[p], sem_g).wait()

            @pl.when(g + 1 < NCHUNK)
            def _():
                fetch(g + 1, 1 - p)

            pvec = jnp.broadcast_to(p, (16,))
            for k in range(2):
                @pl.when(g > 0)
                def _():
                    pltpu.make_async_copy(
                        out_hbm.at[pl.ds(0, HALF * NCB)],
                        out_v.at[k], sem_o[k]).wait()

                @pl.loop(0, HALF // 16)
                def row_loop(t):
                    rvec = k * HALF + t * 16 + iota16
                    rvec8 = rvec * NCB
                    vals = []
                    for h in range(NCB):
                        hvec = jnp.full((16,), h, jnp.int32)
                        vals.append(
                            plsc.load_gather(il_v, [pvec, rvec, hvec]))
                    for h in range(NCB):
                        plsc.store_scatter(fl_v, [rvec8 + h], vals[h])

                for j in range(G2):
                    pltpu.async_copy(
                        cb_sh.at[fl_v.at[pl.ds(k * HALF * NCB + j * FSUB,
                                               FSUB)]],
                        out_v.at[k, pl.ds(j * FSUB, FSUB)],
                        sem_c[k])

            for k in range(2):
                pltpu.make_async_copy(out_hbm.at[pl.ds(0, HALF * NCB)],
                                      out_v.at[k], sem_c[k]).wait()
                out_off = pl.multiple_of(
                    (row_base + g * CHUNK + k * HALF) * NCB, 8)
                pltpu.async_copy(out_v.at[k],
                                 out_hbm.at[pl.ds(out_off, HALF * NCB)],
                                 sem_o[k])

        for k in range(2):
            pltpu.make_async_copy(out_hbm.at[pl.ds(0, HALF * NCB)],
                                  out_v.at[k], sem_o[k]).wait()

    return rpq_sc


_RPQ_SC = _build_kernel()


@jax.jit
def kernel(input, codes, codebooks):
    idx = input.reshape(N // SUB, SUB)
    cb2 = codebooks.reshape(NCB * CB_SIZE, CB_DIM)
    out, _ = _RPQ_SC(idx, codes, cb2)
    return out.reshape(input.shape + (DIM,))


# speedup vs baseline: 1.2576x; 1.2576x over previous
import functools

import jax
import jax.numpy as jnp
from jax import lax
from jax.experimental import pallas as pl
from jax.experimental.pallas import tpu as pltpu
from jax.experimental.pallas import tpu_sc as plsc

N_EMB = 1000000
DIM = 64
NCB = 8
CB_SIZE = 256
CB_DIM = 8
BATCH = 4096
HIST = 200
N = BATCH * HIST

NW = 32
NSUBC = 16
PER_W = N // NW
SUB = 128
CHUNK = 1024
HALF = CHUNK // 2
NSUB = CHUNK // SUB
NCHUNK = PER_W // CHUNK
G2 = HALF * NCB // SUB


def _build_main_kernel():
    mesh = plsc.VectorSubcoreMesh(core_axis_name="c", subcore_axis_name="s")

    @functools.partial(
        pl.kernel,
        out_type=jax.ShapeDtypeStruct((N * NCB, CB_DIM), jnp.float32),
        mesh=mesh,
        scratch_types=[
            pltpu.VMEM((2, NSUB, SUB), jnp.int32),
            pltpu.VMEM((2, CHUNK, NCB), jnp.int32),
            pltpu.VMEM((CHUNK * NCB,), jnp.int32),
            pltpu.VMEM((2, HALF * NCB, CB_DIM), jnp.float32),
            pltpu.VMEM_SHARED((NCB * CB_SIZE, CB_DIM), jnp.float32),
            pltpu.SemaphoreType.DMA,
            pltpu.SemaphoreType.DMA,
            pltpu.SemaphoreType.DMA,
            pltpu.SemaphoreType.DMA,
            pltpu.SemaphoreType.DMA,
        ],
        compiler_params=pltpu.CompilerParams(
            needs_layout_passes=False, use_tc_tiling_on_sc=False),
    )
    def rpq_sc(idx_hbm, codes_t_hbm, cb_hbm, out_hbm,
               idx_v, il_v, fl_v, out_v, cb_sh,
               sem_g, sem_c0, sem_c1, sem_o0, sem_o1):
        wid = lax.axis_index("c") * NSUBC + lax.axis_index("s")
        row_base = wid * PER_W
        sub_base = row_base // SUB
        sem_c = (sem_c0, sem_c1)
        sem_o = (sem_o0, sem_o1)
        iota16 = lax.iota(jnp.int32, 16)

        @pl.when(lax.axis_index("s") == 0)
        def _():
            pltpu.sync_copy(cb_hbm, cb_sh)

        plsc.subcore_barrier()

        def fetch(g, slot):
            sub_off = pl.multiple_of(sub_base + g * NSUB, 8)
            pltpu.sync_copy(idx_hbm.at[pl.ds(sub_off, NSUB)], idx_v.at[slot])
            for j in range(NSUB):
                pltpu.async_copy(codes_t_hbm.at[idx_v.at[slot, j]],
                                 il_v.at[slot, pl.ds(j * SUB, SUB)],
                                 sem_g)

        fetch(0, 0)

        @pl.loop(0, NCHUNK)
        def chunk_loop(g):
            p = lax.rem(g, 2)
            pltpu.make_async_copy(codes_t_hbm.at[pl.ds(0, CHUNK)],
                                  il_v.at[p], sem_g).wait()

            @pl.when(g + 1 < NCHUNK)
            def _():
                fetch(g + 1, 1 - p)

            pvec = jnp.broadcast_to(p, (16,))
            for k in range(2):
                @pl.when(g > 0)
                def _():
                    pltpu.make_async_copy(
                        out_hbm.at[pl.ds(0, HALF * NCB)],
                        out_v.at[k], sem_o[k]).wait()

                @pl.loop(0, HALF // 16)
                def row_loop(t):
                    rvec = k * HALF + t * 16 + iota16
                    rvec8 = rvec * NCB
                    vals = []
                    for h in range(NCB):
                        hvec = jnp.full((16,), h, jnp.int32)
                        vals.append(
                            plsc.load_gather(il_v, [pvec, rvec, hvec])
                            + (h * CB_SIZE))
                    for h in range(NCB):
                        plsc.store_scatter(fl_v, [rvec8 + h], vals[h])

                for j in range(G2):
                    pltpu.async_copy(
                        cb_sh.at[fl_v.at[pl.ds(k * HALF * NCB + j * SUB,
                                               SUB)]],
                        out_v.at[k, pl.ds(j * SUB, SUB)],
                        sem_c[k])

            for k in range(2):
                pltpu.make_async_copy(out_hbm.at[pl.ds(0, HALF * NCB)],
                                      out_v.at[k], sem_c[k]).wait()
                out_off = pl.multiple_of(
                    (row_base + g * CHUNK + k * HALF) * NCB, 8)
                pltpu.async_copy(out_v.at[k],
                                 out_hbm.at[pl.ds(out_off, HALF * NCB)],
                                 sem_o[k])

        for k in range(2):
            pltpu.make_async_copy(out_hbm.at[pl.ds(0, HALF * NCB)],
                                  out_v.at[k], sem_o[k]).wait()

    return rpq_sc


_RPQ_SC = _build_main_kernel()


@jax.jit
def kernel(input, codes, codebooks):
    idx = input.reshape(N // SUB, SUB)
    codes_t = jnp.swapaxes(codes, 0, 1)
    cb2 = codebooks.reshape(NCB * CB_SIZE, CB_DIM)
    out = _RPQ_SC(idx, codes_t, cb2)
    return out.reshape(input.shape + (DIM,))
